# baseline (speedup 1.0000x reference)
import jax
import jax.numpy as jnp
from jax.experimental import pallas as pl
from jax.experimental.pallas import tpu as pltpu

B, V, D = 64, 101, 64
VP = 104
K = 30
EPS = 1e-5
NTOT = float(B * V * V)
DIMS = [64, 32, 32, 16, 16]


def _lrelu(v):
    return jnp.where(v >= 0, v, 0.01 * v)


def _phi_rows(x_ref):
    xb = x_ref[0]
    xjp = jnp.concatenate([xb, jnp.zeros((VP - V, D), jnp.float32)], axis=0)
    phi3 = jnp.abs(xb[:, None, :] - xjp[None, :, :])
    phi = phi3.reshape(V * VP, D)
    rm3 = jax.lax.broadcasted_iota(jnp.int32, (V, VP, 1), 1) < V
    rmf = rm3.reshape(V * VP, 1).astype(jnp.float32)
    return phi, rmf


def _conv(z, wref, bref):
    return jax.lax.dot_general(
        z.astype(jnp.bfloat16), wref[...].astype(jnp.bfloat16),
        (((1,), (1,)), ((), ())),
        preferred_element_type=jnp.float32) + bref[...]


def _bn_params(sref, c, gref, beref):
    mean = sref[0:1, :c] / NTOT
    var = sref[1:2, :c] / NTOT - mean * mean
    rstd = jax.lax.rsqrt(var + EPS)
    scale = gref[...] * rstd
    shift = beref[...] - mean * scale
    return scale, shift


def _stats_body(x_ref, w0, bb0, g0, be0, w1, bb1, g1, be1, w2, bb2, g2, be2,
                w3, bb3, g3, be3, w4, bb4, s1, s2, s3, s4):
    p = pl.program_id(0)
    b = pl.program_id(1)
    ws = [w0, w1, w2, w3]
    bs = [bb0, bb1, bb2, bb3]
    gs = [g0, g1, g2, g3]
    bes = [be0, be1, be2, be3]
    srefs = [s1, s2, s3, s4]

    phi, rmf = _phi_rows(x_ref)

    for ph in range(4):
        @pl.when(p == ph)
        def _(ph=ph):
            z = phi
            for k in range(ph):
                y = _conv(z, ws[k], bs[k])
                scale, shift = _bn_params(srefs[k], DIMS[k + 1], gs[k], bes[k])
                z = _lrelu(y * scale + shift)
            y = _conv(z, ws[ph], bs[ph])

            sref = srefs[ph]

            @pl.when(b == 0)
            def _():
                sref[0:2, :] = jnp.zeros((2, 128), jnp.float32)

            c = DIMS[ph + 1]
            ym = y * rmf
            sref[0:1, :c] += jnp.sum(ym, axis=0, keepdims=True)
            sref[1:2, :c] += jnp.sum(y * ym, axis=0, keepdims=True)


def _final_body(x_ref, w0, bb0, g0, be0, w1, bb1, g1, be1, w2, bb2, g2, be2,
                w3, bb3, g3, be3, w4, bb4, s1, s2, s3, s4, out_ref):
    ws = [w0, w1, w2, w3]
    bs = [bb0, bb1, bb2, bb3]
    gs = [g0, g1, g2, g3]
    bes = [be0, be1, be2, be3]
    srefs = [s1, s2, s3, s4]

    phi, _ = _phi_rows(x_ref)
    z = phi
    for k in range(4):
        y = _conv(z, ws[k], bs[k])
        scale, shift = _bn_params(srefs[k], DIMS[k + 1], gs[k], bes[k])
        z = _lrelu(y * scale + shift)
    z3 = z.reshape(V, VP, DIMS[4]).astype(jnp.bfloat16).astype(jnp.float32)
    w4v = w4[...].reshape(1, 1, DIMS[4]).astype(jnp.bfloat16).astype(jnp.float32)
    logits = jnp.sum(z3 * w4v, axis=-1) + bb4[...]

    jm = jax.lax.broadcasted_iota(jnp.int32, (V, VP), 1) < V
    lm = jnp.where(jm, logits, jnp.float32(-1e30))
    lmax = jnp.max(lm, axis=-1, keepdims=True)
    e = jnp.exp(lm - lmax) * jm.astype(jnp.float32)
    soft = e / jnp.sum(e, axis=-1, keepdims=True)

    iota = jax.lax.broadcasted_iota(jnp.int32, (V, VP), 1)
    work = jnp.where(jm, soft, -1.0)
    mask = jnp.zeros((V, VP), jnp.float32)
    for _k in range(K):
        m = jnp.max(work, axis=-1, keepdims=True)
        cand = work == m
        idx = jnp.min(jnp.where(cand, iota, VP + 1), axis=-1, keepdims=True)
        first = iota == idx
        mask = jnp.where(first, 1.0, mask)
        work = jnp.where(first, -1.0, work)

    out_ref[0] = (soft * mask)[:, :V]


@jax.jit
def kernel(x, W0, b0, g0, be0, W1, b1, g1, be1, W2, b2, g2, be2, W3, b3, g3, be3, W4, b4):
    vec = lambda v: v.reshape(1, -1)
    args = [x,
            W0, vec(b0), vec(g0), vec(be0),
            W1, vec(b1), vec(g1), vec(be1),
            W2, vec(b2), vec(g2), vec(be2),
            W3, vec(b3), vec(g3), vec(be3),
            W4, vec(b4)]

    stat_shape = jax.ShapeDtypeStruct((8, 128), jnp.float32)

    full2 = lambda a: pl.BlockSpec(a.shape, lambda p, b: (0,) * a.ndim)
    stats = pl.pallas_call(
        _stats_body,
        grid=(4, B),
        in_specs=[pl.BlockSpec((1, V, D), lambda p, b: (b, 0, 0))]
        + [full2(a) for a in args[1:]],
        out_specs=[pl.BlockSpec((8, 128), lambda p, b: (0, 0))] * 4,
        out_shape=[stat_shape] * 4,
        compiler_params=pltpu.CompilerParams(
            dimension_semantics=("arbitrary", "arbitrary")),
    )(*args)

    full1 = lambda a: pl.BlockSpec(a.shape, lambda b: (0,) * a.ndim)
    return pl.pallas_call(
        _final_body,
        grid=(B,),
        in_specs=[pl.BlockSpec((1, V, D), lambda b: (b, 0, 0))]
        + [full1(a) for a in args[1:]]
        + [pl.BlockSpec((8, 128), lambda b: (0, 0))] * 4,
        out_specs=pl.BlockSpec((1, V, V), lambda b: (b, 0, 0)),
        out_shape=jax.ShapeDtypeStruct((B, V, V), jnp.float32),
        compiler_params=pltpu.CompilerParams(
            dimension_semantics=("arbitrary",)),
    )(*args, *stats)
